# baseline (speedup 1.0000x reference)
import functools

import jax
import jax.numpy as jnp
from jax import lax
from jax.experimental import pallas as pl
from jax.experimental.pallas import tpu as pltpu
from jax.experimental.pallas import tpu_sc as plsc

VOCAB = 1000000
EMB = 64
HID = 128
B = 4096
L = 50
TOK = B * L

NC = 2
NS = 16
NW = NC * NS
ROWS_PER_W = TOK // NW
GRP = 128
CHUNK = 640
N_CHUNKS = ROWS_PER_W // CHUNK


def _sc_gather(table, idx):
    mesh = plsc.VectorSubcoreMesh(core_axis_name="c", subcore_axis_name="s")

    @functools.partial(
        pl.kernel,
        mesh=mesh,
        out_type=jax.ShapeDtypeStruct((TOK, EMB), jnp.float32),
        scratch_types=[
            pltpu.VMEM((ROWS_PER_W,), jnp.int32),
            pltpu.VMEM((CHUNK, EMB), jnp.float32),
            pltpu.SemaphoreType.DMA,
        ],
        compiler_params=pltpu.CompilerParams(use_tc_tiling_on_sc=False),
    )
    def k(table_hbm, idx_hbm, out_hbm, idx_v, buf_v, sem):
        wid = lax.axis_index("s") * NC + lax.axis_index("c")
        base = wid * ROWS_PER_W
        pltpu.sync_copy(idx_hbm.at[pl.ds(base, ROWS_PER_W)], idx_v)

        def chunk_body(s, carry):
            off = pl.multiple_of(s * CHUNK, CHUNK)
            cps = [
                pltpu.async_copy(
                    table_hbm.at[idx_v.at[pl.ds(off + g * GRP, GRP)]],
                    buf_v.at[pl.ds(g * GRP, GRP)],
                    sem,
                )
                for g in range(CHUNK // GRP)
            ]
            for cp in cps:
                cp.wait()
            pltpu.sync_copy(buf_v, out_hbm.at[pl.ds(base + off, CHUNK)])
            return carry

        lax.fori_loop(0, N_CHUNKS, chunk_body, 0)

    return k(table, idx)


BB = 64
TB = BB * L


def _tc_body(emb_ref, mask_ref, w1t_ref, b1_ref, g_ref, bta_ref, wpt_ref,
             bp_ref, out_ref):
    e = emb_ref[...] * mask_ref[...]
    h = jnp.dot(e, w1t_ref[...], preferred_element_type=jnp.float32)
    h = h + b1_ref[...]
    mu = jnp.mean(h, axis=1, keepdims=True)
    d = h - mu
    var = jnp.mean(d * d, axis=1, keepdims=True)
    hn = d * lax.rsqrt(var + 1e-5) * g_ref[...] + bta_ref[...]
    hr = jnp.maximum(hn, 0.0)
    rows = lax.broadcasted_iota(jnp.int32, (BB, TB), 0)
    cols = lax.broadcasted_iota(jnp.int32, (BB, TB), 1)
    pool = jnp.where(cols // L == rows, 1.0 / L, 0.0)
    pooled = jnp.dot(pool, hr, preferred_element_type=jnp.float32)
    o = jnp.dot(pooled, wpt_ref[...], preferred_element_type=jnp.float32)
    o = o + bp_ref[...]
    n2 = jnp.sum(o * o, axis=1, keepdims=True)
    out_ref[...] = o * lax.rsqrt(jnp.maximum(n2, 1e-24))


def _tc_mlp(emb, mask, w1t, b1, ln_g, ln_b, wpt, bp):
    grid = (B // BB,)
    return pl.pallas_call(
        _tc_body,
        grid=grid,
        in_specs=[
            pl.BlockSpec((TB, EMB), lambda i: (i, 0)),
            pl.BlockSpec((TB, 1), lambda i: (i, 0)),
            pl.BlockSpec((EMB, HID), lambda i: (0, 0)),
            pl.BlockSpec((1, HID), lambda i: (0, 0)),
            pl.BlockSpec((1, HID), lambda i: (0, 0)),
            pl.BlockSpec((1, HID), lambda i: (0, 0)),
            pl.BlockSpec((HID, EMB), lambda i: (0, 0)),
            pl.BlockSpec((1, EMB), lambda i: (0, 0)),
        ],
        out_specs=pl.BlockSpec((BB, EMB), lambda i: (i, 0)),
        out_shape=jax.ShapeDtypeStruct((B, EMB), jnp.float32),
    )(emb, mask, w1t, b1, ln_g, ln_b, wpt, bp)


def kernel(x, padding_mask, table, W1, b1, ln_g, ln_b, Wp, bp):
    idx = x.reshape(TOK).astype(jnp.int32)
    gathered = _sc_gather(table, idx)
    mask2d = padding_mask.reshape(TOK, 1)
    return _tc_mlp(
        gathered,
        mask2d,
        W1.T,
        b1.reshape(1, HID),
        ln_g.reshape(1, HID),
        ln_b.reshape(1, HID),
        Wp.T,
        bp.reshape(1, EMB),
    )


# speedup vs baseline: 1.2243x; 1.2243x over previous
import functools

import jax
import jax.numpy as jnp
from jax import lax
from jax.experimental import pallas as pl
from jax.experimental.pallas import tpu as pltpu
from jax.experimental.pallas import tpu_sc as plsc

VOCAB = 1000000
EMB = 64
HID = 128
B = 4096
L = 50
TOK = B * L

TW = 2048


def _tp_body(tt_ref, out_ref):
    t = tt_ref[...].T
    out_ref[...] = jnp.concatenate(
        [t, jnp.zeros((TW, 128 - EMB), jnp.float32)], axis=1)


def _transpose128(tableT):
    return pl.pallas_call(
        _tp_body,
        grid=(pl.cdiv(VOCAB, TW),),
        in_specs=[pl.BlockSpec((EMB, TW), lambda i: (0, i))],
        out_specs=pl.BlockSpec((TW, 128), lambda i: (i, 0)),
        out_shape=jax.ShapeDtypeStruct((VOCAB, 128), jnp.float32),
    )(tableT)


NC = 2
NS = 16
NW = NC * NS
ROWS_PER_W = TOK // NW
GRP = 128
CHUNK = 640
N_CHUNKS = ROWS_PER_W // CHUNK


def _sc_gather(t128, idx):
    mesh = plsc.VectorSubcoreMesh(core_axis_name="c", subcore_axis_name="s")

    @functools.partial(
        pl.kernel,
        mesh=mesh,
        out_type=jax.ShapeDtypeStruct((TOK, 2 * EMB), jnp.float32),
        scratch_types=[
            pltpu.VMEM((ROWS_PER_W,), jnp.int32),
            pltpu.VMEM((CHUNK, 2 * EMB), jnp.float32),
            pltpu.SemaphoreType.DMA,
        ],
        compiler_params=pltpu.CompilerParams(use_tc_tiling_on_sc=False),
    )
    def k(t_hbm, idx_hbm, out_hbm, idx_v, buf_v, sem):
        wid = lax.axis_index("s") * NC + lax.axis_index("c")
        base = wid * ROWS_PER_W
        pltpu.sync_copy(idx_hbm.at[pl.ds(base, ROWS_PER_W)], idx_v)

        def chunk_body(s, carry):
            off = pl.multiple_of(s * CHUNK, CHUNK)
            cps = [
                pltpu.async_copy(
                    t_hbm.at[idx_v.at[pl.ds(off + g * GRP, GRP)]],
                    buf_v.at[pl.ds(g * GRP, GRP)],
                    sem,
                )
                for g in range(CHUNK // GRP)
            ]
            for cp in cps:
                cp.wait()
            pltpu.sync_copy(buf_v, out_hbm.at[pl.ds(base + off, CHUNK)])
            return carry

        lax.fori_loop(0, N_CHUNKS, chunk_body, 0)

    return k(t128, idx)


BB = 64
TB = BB * L


def _tc_body(emb_ref, mask_ref, w1t_ref, b1_ref, g_ref, bta_ref, wpt_ref,
             bp_ref, out_ref, pool_ref):
    i = pl.program_id(0)

    @pl.when(i == 0)
    def _():
        rows = lax.broadcasted_iota(jnp.int32, (BB, TB), 0)
        cols = lax.broadcasted_iota(jnp.int32, (BB, TB), 1)
        pool_ref[...] = jnp.where(cols // L == rows, 1.0 / L, 0.0)

    e = emb_ref[...] * mask_ref[...]
    h = jnp.dot(e, w1t_ref[...], preferred_element_type=jnp.float32)
    h = h + b1_ref[...]
    ones_h = jnp.full((HID, 1), 1.0 / HID, dtype=jnp.float32)
    mu = jnp.dot(h, ones_h, preferred_element_type=jnp.float32)
    m2 = jnp.dot(h * h, ones_h, preferred_element_type=jnp.float32)
    inv = lax.rsqrt(m2 - mu * mu + 1e-5)
    hn = (h - mu) * inv * g_ref[...] + bta_ref[...]
    hr = jnp.maximum(hn, 0.0)
    pooled = jnp.dot(pool_ref[...], hr, preferred_element_type=jnp.float32)
    o = jnp.dot(pooled, wpt_ref[...], preferred_element_type=jnp.float32)
    o = o + bp_ref[...]
    n2 = jnp.sum(o * o, axis=1, keepdims=True)
    out_ref[...] = o * lax.rsqrt(jnp.maximum(n2, 1e-24))


def _tc_mlp(emb, mask, w1t, b1, ln_g, ln_b, wpt, bp):
    return pl.pallas_call(
        _tc_body,
        grid=(B // BB,),
        in_specs=[
            pl.BlockSpec((TB, 2 * EMB), lambda i: (i, 0)),
            pl.BlockSpec((TB, 1), lambda i: (i, 0)),
            pl.BlockSpec((2 * EMB, HID), lambda i: (0, 0)),
            pl.BlockSpec((1, HID), lambda i: (0, 0)),
            pl.BlockSpec((1, HID), lambda i: (0, 0)),
            pl.BlockSpec((1, HID), lambda i: (0, 0)),
            pl.BlockSpec((HID, EMB), lambda i: (0, 0)),
            pl.BlockSpec((1, EMB), lambda i: (0, 0)),
        ],
        out_specs=pl.BlockSpec((BB, EMB), lambda i: (i, 0)),
        out_shape=jax.ShapeDtypeStruct((B, EMB), jnp.float32),
        scratch_shapes=[pltpu.VMEM((BB, TB), jnp.float32)],
    )(emb, mask, w1t, b1, ln_g, ln_b, wpt, bp)


def kernel(x, padding_mask, table, W1, b1, ln_g, ln_b, Wp, bp):
    idx = x.reshape(TOK).astype(jnp.int32)
    t128 = _transpose128(table.T)
    gathered = _sc_gather(t128, idx)
    mask2d = padding_mask.reshape(TOK, 1)
    w1t_pad = jnp.pad(W1.T, ((0, EMB), (0, 0)))
    return _tc_mlp(
        gathered,
        mask2d,
        w1t_pad,
        b1.reshape(1, HID),
        ln_g.reshape(1, HID),
        ln_b.reshape(1, HID),
        Wp.T,
        bp.reshape(1, EMB),
    )
